# baseline (speedup 1.0000x reference)
import numpy as np
import jax
import jax.numpy as jnp
from jax import lax
from jax.experimental import pallas as pl
from jax.experimental.pallas import tpu as pltpu

_EPS = 1e-06


def _softplus(x):
    return jnp.maximum(x, 0.0) + jnp.log1p(jnp.exp(-jnp.abs(x)))


def _make_kernel(N, D, HIDDEN, RANK, TI, TJ, T):
    inv_count = 1.0 / (N + 1)

    def body(im_ref, jm_ref,
             rf_ref, attn_ref, W1a_ref, W1b_ref, w1c_ref, b1_ref,
             W2_ref, b2_ref, W3_ref, b3_ref, Wd_ref, bd_ref,
             GrF_ref, GrT_ref, GcF_ref, GcT_ref,
             sigma_ref, A_s, C_s, S_s):
        t = pl.program_id(1)
        i = im_ref[t]
        j = jm_ref[t]

        @pl.when(t == 0)
        def _init():
            rf = rf_ref[0]
            A_s[...] = jnp.dot(rf, W1a_ref[...],
                               preferred_element_type=jnp.float32) + b1_ref[0]
            C_s[...] = jnp.dot(rf, W1b_ref[...],
                               preferred_element_type=jnp.float32)
            S_s[...] = jnp.zeros((N, HIDDEN), jnp.float32)

        a = A_s[pl.ds(i * TI, TI), :]
        c = C_s[pl.ds(j * TJ, TJ), :]
        t1 = attn_ref[0][:, :, None] * w1c_ref[0][None, None, :]
        x1 = jnp.maximum(t1 + a[:, None, :] + c[None, :, :], 0.0)
        x2 = jnp.maximum(
            jnp.dot(x1.reshape(TI * TJ, HIDDEN), W2_ref[...],
                    preferred_element_type=jnp.float32)
            + b2_ref[0], 0.0)

        def _accumulate(gr_ref, gc_ref):
            S_s[pl.ds(i * TI, TI), :] += jnp.dot(
                gr_ref[...], x2, preferred_element_type=jnp.float32)
            S_s[pl.ds(j * TJ, TJ), :] += jnp.dot(
                gc_ref[...], x2, preferred_element_type=jnp.float32)

        @pl.when(i == j)
        def _diag_tile():
            _accumulate(GrT_ref, GcT_ref)

        @pl.when(i != j)
        def _full_tile():
            _accumulate(GrF_ref, GcF_ref)

        @pl.when(t == T - 1)
        def _finish():
            u = (jnp.dot(S_s[...], W3_ref[...],
                         preferred_element_type=jnp.float32) * inv_count
                 + b3_ref[0])
            sig = lax.dot_general(u, u, (((1,), (1,)), ((), ())),
                                  preferred_element_type=jnp.float32)
            rf = rf_ref[0]
            d_raw = jnp.dot(rf, Wd_ref[...],
                            preferred_element_type=jnp.float32)[:, 0]
            d = _softplus(d_raw + bd_ref[0, 0]) + _EPS
            rr = lax.broadcasted_iota(jnp.int32, (N, N), 0)
            cc = lax.broadcasted_iota(jnp.int32, (N, N), 1)
            sigma_ref[0] = sig + jnp.where(rr == cc, d[:, None], 0.0)

    return body


def kernel(residue_features, attention, W1, b1, W2, b2, W3, b3, Wd, bd):
    B, N, D = residue_features.shape
    HIDDEN = W2.shape[0]
    RANK = W3.shape[1]
    TI = TJ = 128
    nI = N // TI
    nJ = N // TJ

    tril = [(i, j) for i in range(nI) for j in range(nJ) if j <= i]
    T = len(tril)
    imap = jnp.asarray(np.array([p[0] for p in tril], np.int32))
    jmap = jnp.asarray(np.array([p[1] for p in tril], np.int32))

    r = np.arange(TI * TJ)
    il = (r // TJ)[None, :]
    jl = (r % TJ)[None, :]
    sel = np.arange(TI)[:, None]
    GrF = (il == sel).astype(np.float32)
    GrT = ((il == sel) & (jl <= sel)).astype(np.float32)
    GcF = (jl == sel).astype(np.float32)
    GcT = ((jl == sel) & (il >= sel)).astype(np.float32)

    W1a = W1[:D]
    W1b = W1[D:2 * D]
    w1c = W1[2 * D].reshape(1, HIDDEN)
    b1r = b1.reshape(1, HIDDEN)
    b2r = b2.reshape(1, HIDDEN)
    b3r = b3.reshape(1, RANK)
    bdr = bd.reshape(1, 1)

    body = _make_kernel(N, D, HIDDEN, RANK, TI, TJ, T)

    full = lambda *shape: pl.BlockSpec(
        shape, lambda b, t, im, jm: (0,) * len(shape))

    grid_spec = pltpu.PrefetchScalarGridSpec(
        num_scalar_prefetch=2,
        grid=(B, T),
        in_specs=[
            pl.BlockSpec((1, N, D), lambda b, t, im, jm: (b, 0, 0)),
            pl.BlockSpec((1, TI, TJ),
                         lambda b, t, im, jm: (b, im[t], jm[t])),
            full(D, HIDDEN),
            full(D, HIDDEN),
            full(1, HIDDEN),
            full(1, HIDDEN),
            full(HIDDEN, HIDDEN),
            full(1, HIDDEN),
            full(HIDDEN, RANK),
            full(1, RANK),
            full(D, 1),
            full(1, 1),
            full(TI, TI * TJ),
            full(TI, TI * TJ),
            full(TJ, TI * TJ),
            full(TJ, TI * TJ),
        ],
        out_specs=pl.BlockSpec((1, N, N), lambda b, t, im, jm: (b, 0, 0)),
        scratch_shapes=[
            pltpu.VMEM((N, HIDDEN), jnp.float32),
            pltpu.VMEM((N, HIDDEN), jnp.float32),
            pltpu.VMEM((N, HIDDEN), jnp.float32),
        ],
    )

    out = pl.pallas_call(
        body,
        grid_spec=grid_spec,
        out_shape=jax.ShapeDtypeStruct((B, N, N), jnp.float32),
        compiler_params=pltpu.CompilerParams(
            dimension_semantics=("arbitrary", "arbitrary"),
        ),
    )(imap, jmap,
      residue_features, attention, W1a, W1b, w1c, b1r,
      W2, b2r, W3, b3r, Wd, bdr,
      jnp.asarray(GrF), jnp.asarray(GrT), jnp.asarray(GcF), jnp.asarray(GcT))
    return out


# speedup vs baseline: 1.5251x; 1.5251x over previous
import numpy as np
import jax
import jax.numpy as jnp
from jax import lax
from jax.experimental import pallas as pl
from jax.experimental.pallas import tpu as pltpu

_EPS = 1e-06


def _softplus(x):
    return jnp.maximum(x, 0.0) + jnp.log1p(jnp.exp(-jnp.abs(x)))


def _make_kernel(N, D, HIDDEN, RANK, TI, TJ, T):
    inv_count = 1.0 / (N + 1)
    TF = TI // 2
    L = 2 * HIDDEN

    def body(im_ref, jm_ref,
             rf_ref, attn_ref, W1a_ref, W1b_ref, b1_ref,
             w1cL_ref, w1cR_ref, W2bd_ref, b2d_ref,
             W3_ref, W3bd_ref, b3_ref, Wd_ref, bd_ref,
             PL_ref, PR_ref, m3_ref,
             sigma_ref, A_s, C_s, S_s, R8_s):
        t = pl.program_id(1)
        i = im_ref[t]
        j = jm_ref[t]

        @pl.when(t == 0)
        def _init():
            rf = rf_ref[0]
            A_s[...] = jnp.dot(rf, W1a_ref[...],
                               preferred_element_type=jnp.float32) + b1_ref[0]
            C_s[...] = jnp.dot(rf, W1b_ref[...],
                               preferred_element_type=jnp.float32)
            S_s[...] = jnp.zeros((N, HIDDEN), jnp.float32)
            R8_s[...] = jnp.zeros((N // 2, 8, L), jnp.float32)

        a2 = jnp.concatenate([A_s[pl.ds(i * TI, TF), :],
                              A_s[pl.ds(i * TI + TF, TF), :]], axis=1)
        cd = jnp.concatenate([C_s[pl.ds(j * TJ, TJ), :],
                              C_s[pl.ds(j * TJ, TJ), :]], axis=1)
        attn = attn_ref[0]
        aE = attn[0:TF, :]
        aO = attn[TF:TI, :]
        x1 = jnp.maximum(
            aE[:, :, None] * w1cL_ref[0][None, None, :]
            + aO[:, :, None] * w1cR_ref[0][None, None, :]
            + a2[:, None, :] + cd[None, :, :], 0.0)
        x2 = jnp.maximum(
            jnp.dot(x1.reshape(TF * TJ, L), W2bd_ref[...],
                    preferred_element_type=jnp.float32)
            + b2d_ref[0], 0.0).reshape(TF, TJ, L)

        def _accumulate(get_tile):
            for ic in range(TF // 8):
                racc = None
                for jc in range(TJ // 8):
                    tk = get_tile(ic, jc)
                    racc = tk if racc is None else racc + tk
                    cf = jnp.sum(tk, axis=0)
                    S_s[pl.ds(j * TJ + jc * 8, 8), :] += (
                        cf[:, 0:HIDDEN] + cf[:, HIDDEN:L])
                R8_s[pl.ds(i * TF + ic * 8, 8), :, :] += racc

        @pl.when(i == j)
        def _diag_tile():
            _accumulate(lambda ic, jc: x2[8 * ic:8 * ic + 8,
                                          8 * jc:8 * jc + 8, :]
                        * m3_ref[8 * ic:8 * ic + 8, 8 * jc:8 * jc + 8, :])

        @pl.when(i != j)
        def _full_tile():
            _accumulate(lambda ic, jc: x2[8 * ic:8 * ic + 8,
                                          8 * jc:8 * jc + 8, :])

        @pl.when(t == T - 1)
        def _finish():
            rrow = jnp.sum(R8_s[...], axis=1)
            uf = jnp.dot(rrow, W3bd_ref[...],
                         preferred_element_type=jnp.float32)
            u_row = (jnp.dot(PL_ref[...], uf[:, 0:RANK],
                             preferred_element_type=jnp.float32)
                     + jnp.dot(PR_ref[...], uf[:, RANK:2 * RANK],
                               preferred_element_type=jnp.float32))
            u_col = jnp.dot(S_s[...], W3_ref[...],
                            preferred_element_type=jnp.float32)
            u = (u_row + u_col) * inv_count + b3_ref[0]
            sig = lax.dot_general(u, u, (((1,), (1,)), ((), ())),
                                  preferred_element_type=jnp.float32)
            rf = rf_ref[0]
            d_raw = jnp.dot(rf, Wd_ref[...],
                            preferred_element_type=jnp.float32)[:, 0]
            d = _softplus(d_raw + bd_ref[0, 0]) + _EPS
            rr = lax.broadcasted_iota(jnp.int32, (N, N), 0)
            cc = lax.broadcasted_iota(jnp.int32, (N, N), 1)
            sigma_ref[0] = sig + jnp.where(rr == cc, d[:, None], 0.0)

    return body


def kernel(residue_features, attention, W1, b1, W2, b2, W3, b3, Wd, bd):
    B, N, D = residue_features.shape
    HIDDEN = W2.shape[0]
    RANK = W3.shape[1]
    TI = TJ = 128
    TF = TI // 2
    L = 2 * HIDDEN
    nI = N // TI

    tril = [(i, j) for i in range(nI) for j in range(nI) if j <= i]
    T = len(tril)
    imap = jnp.asarray(np.array([p[0] for p in tril], np.int32))
    jmap = jnp.asarray(np.array([p[1] for p in tril], np.int32))

    W1a = W1[:D]
    W1b = W1[D:2 * D]
    w1c = W1[2 * D]
    z = jnp.zeros((HIDDEN,), jnp.float32)
    w1cL = jnp.concatenate([w1c, z]).reshape(1, L)
    w1cR = jnp.concatenate([z, w1c]).reshape(1, L)
    zz = jnp.zeros((HIDDEN, HIDDEN), jnp.float32)
    W2bd = jnp.concatenate(
        [jnp.concatenate([W2, zz], axis=1),
         jnp.concatenate([zz, W2], axis=1)], axis=0)
    b2d = jnp.concatenate([b2, b2]).reshape(1, L)
    zr = jnp.zeros((HIDDEN, RANK), jnp.float32)
    W3bd = jnp.concatenate(
        [jnp.concatenate([W3, zr], axis=1),
         jnp.concatenate([zr, W3], axis=1)], axis=0)
    b1r = b1.reshape(1, HIDDEN)
    b3r = b3.reshape(1, RANK)
    bdr = bd.reshape(1, 1)

    k = np.arange(N // 2)
    orig_l = (k // TF) * TI + (k % TF)
    n_idx = np.arange(N)[:, None]
    PL = (n_idx == orig_l[None, :]).astype(np.float32)
    PR = (n_idx == (orig_l + TF)[None, :]).astype(np.float32)

    ii = np.arange(TF)[:, None, None]
    jl = np.arange(TJ)[None, :, None]
    lane = np.arange(L)[None, None, :]
    m3 = np.where(lane < HIDDEN, jl <= ii, jl <= ii + TF).astype(np.float32)

    body = _make_kernel(N, D, HIDDEN, RANK, TI, TJ, T)

    full = lambda *shape: pl.BlockSpec(
        shape, lambda b, t, im, jm: (0,) * len(shape))

    grid_spec = pltpu.PrefetchScalarGridSpec(
        num_scalar_prefetch=2,
        grid=(B, T),
        in_specs=[
            pl.BlockSpec((1, N, D), lambda b, t, im, jm: (b, 0, 0)),
            pl.BlockSpec((1, TI, TJ),
                         lambda b, t, im, jm: (b, im[t], jm[t])),
            full(D, HIDDEN),
            full(D, HIDDEN),
            full(1, HIDDEN),
            full(1, L),
            full(1, L),
            full(L, L),
            full(1, L),
            full(HIDDEN, RANK),
            full(L, 2 * RANK),
            full(1, RANK),
            full(D, 1),
            full(1, 1),
            full(N, N // 2),
            full(N, N // 2),
            full(TF, TJ, L),
        ],
        out_specs=pl.BlockSpec((1, N, N), lambda b, t, im, jm: (b, 0, 0)),
        scratch_shapes=[
            pltpu.VMEM((N, HIDDEN), jnp.float32),
            pltpu.VMEM((N, HIDDEN), jnp.float32),
            pltpu.VMEM((N, HIDDEN), jnp.float32),
            pltpu.VMEM((N // 2, 8, L), jnp.float32),
        ],
    )

    out = pl.pallas_call(
        body,
        grid_spec=grid_spec,
        out_shape=jax.ShapeDtypeStruct((B, N, N), jnp.float32),
        compiler_params=pltpu.CompilerParams(
            dimension_semantics=("arbitrary", "arbitrary"),
        ),
    )(imap, jmap,
      residue_features, attention, W1a, W1b, b1r,
      w1cL, w1cR, W2bd, b2d, W3, W3bd, b3r, Wd, bdr,
      jnp.asarray(PL), jnp.asarray(PR), jnp.asarray(m3))
    return out
